# baseline (speedup 1.0000x reference)
import jax
import jax.numpy as jnp
import numpy as np
from jax import lax
from jax.experimental import pallas as pl
from jax.experimental.pallas import tpu as pltpu
from jax.experimental.pallas import tpu_sc as plsc

_B = 64
_EMBED = 768
_NVIS = 256
_NMASK = 768
_NROWS = 1025
_NW = 32
_RPW = 33
_SP = 48


def _build_maps(sidx_ref, midx_ref, src_ref):
    def init(k, _):
        w = k // _SP
        i = k % _SP
        r = w * _RPW + i
        ok = jnp.logical_and(i < _RPW, r < _NROWS)
        src_ref[w, i] = jnp.where(ok, -1, -3)
        return 0

    lax.fori_loop(0, _NW * _SP, init, 0)
    src_ref[0, 0] = 0

    def samp(j, _):
        r = sidx_ref[j]
        src_ref[r // _RPW, r % _RPW] = j + 1
        return 0

    lax.fori_loop(0, _NVIS, samp, 0)

    def msk(j, _):
        r = midx_ref[j]
        src_ref[r // _RPW, r % _RPW] = -2
        return 0

    lax.fori_loop(0, _NMASK, msk, 0)


def _sc_body(x_hbm, src_hbm, w0_hbm, b_hbm, out_hbm,
             srcb, qbuf, gbuf, xbuf, sbuf, zbuf, wbuf, bbuf, accb, semg, sems):
    cid = lax.axis_index("c")
    sid = lax.axis_index("s")
    wid = sid * 2 + cid
    pltpu.sync_copy(src_hbm.at[wid], srcb)
    pltpu.sync_copy(w0_hbm, wbuf)
    pltpu.sync_copy(b_hbm.at[pl.ds(0, 16)], bbuf)
    acc = jnp.zeros((16,), jnp.float32)
    for k in range(_EMBED // 16):
        acc = acc + wbuf[pl.ds(k * 16, 16)]
    lanes = lax.iota(jnp.int32, 16)
    wsum = acc[0]
    for k in range(1, 16):
        wsum = wsum + acc[k]
    b0 = bbuf[...][0]
    s_val = wsum * np.float32(127.0 / 255.0) + b0
    sv = jnp.full((16,), s_val, jnp.float32)
    zv = jnp.zeros((16,), jnp.float32)

    def fill(row, _):
        for k in range(_EMBED // 16):
            sbuf[row, pl.ds(k * 16, 16)] = sv
            zbuf[row, pl.ds(k * 16, 16)] = zv
        return 0

    lax.fori_loop(0, 32, fill, 0)

    def row_body(i, _):
        src_i = srcb[pl.ds(i, 16)][0]
        r = wid * _RPW + i

        for h in range(2):
            for k in range(2):
                qbuf[pl.ds(k * 16, 16)] = (lanes + (h * 32 + k * 16)) * _NROWS + r

            @pl.when(src_i >= 0)
            def _(h=h):
                for k in range(2):
                    gbuf[pl.ds(k * 16, 16)] = (
                        (lanes + (h * 32 + k * 16)) * (_NVIS + 1) + src_i
                    )
                pltpu.async_copy(x_hbm.at[gbuf], xbuf, semg).wait()
                pltpu.async_copy(xbuf, out_hbm.at[qbuf], sems).wait()

            @pl.when(src_i == -1)
            def _():
                pltpu.async_copy(zbuf, out_hbm.at[qbuf], sems).wait()

            @pl.when(src_i == -2)
            def _():
                pltpu.async_copy(sbuf, out_hbm.at[qbuf], sems).wait()

        return 0

    lax.fori_loop(0, _RPW, row_body, 0)


def kernel(x, sample_index, mask_index, W, b):
    src2d = pl.pallas_call(
        _build_maps,
        in_specs=[
            pl.BlockSpec(memory_space=pltpu.SMEM),
            pl.BlockSpec(memory_space=pltpu.SMEM),
        ],
        out_specs=pl.BlockSpec(memory_space=pltpu.SMEM),
        out_shape=jax.ShapeDtypeStruct((_NW, _SP), jnp.int32),
    )(sample_index, mask_index)

    x2d = jnp.reshape(x, (_B * (1 + _NVIS), _EMBED))
    w0 = jnp.reshape(W[0], (_EMBED,))

    mesh = plsc.VectorSubcoreMesh(core_axis_name="c", subcore_axis_name="s")
    out2 = pl.kernel(
        _sc_body,
        out_type=jax.ShapeDtypeStruct((_B * _NROWS, _EMBED), jnp.float32),
        mesh=mesh,
        scratch_types=[
            pltpu.VMEM((_SP,), jnp.int32),
            pltpu.VMEM((32,), jnp.int32),
            pltpu.VMEM((32,), jnp.int32),
            pltpu.VMEM((32, _EMBED), jnp.float32),
            pltpu.VMEM((32, _EMBED), jnp.float32),
            pltpu.VMEM((32, _EMBED), jnp.float32),
            pltpu.VMEM((_EMBED,), jnp.float32),
            pltpu.VMEM((16,), jnp.float32),
            pltpu.VMEM((16,), jnp.float32),
            pltpu.SemaphoreType.DMA,
            pltpu.SemaphoreType.DMA,
        ],
    )(x2d, src2d, w0, b)

    return jnp.reshape(out2, (_B, _NROWS, _EMBED))


# speedup vs baseline: 1.0104x; 1.0104x over previous
import jax
import jax.numpy as jnp
import numpy as np
from jax import lax
from jax.experimental import pallas as pl
from jax.experimental.pallas import tpu as pltpu
from jax.experimental.pallas import tpu_sc as plsc

_B = 64
_EMBED = 768
_NVIS = 256
_NMASK = 768
_NROWS = 1025
_NW = 32
_RPW = 33
_SP = 48


def _build_maps(sidx_ref, midx_ref, src_ref):
    def init(k, _):
        w = k // _SP
        i = k % _SP
        r = w * _RPW + i
        ok = jnp.logical_and(i < _RPW, r < _NROWS)
        src_ref[w, i] = jnp.where(ok, -1, -3)
        return 0

    lax.fori_loop(0, _NW * _SP, init, 0)
    src_ref[0, 0] = 0

    def samp(j, _):
        r = sidx_ref[j]
        src_ref[r // _RPW, r % _RPW] = j + 1
        return 0

    lax.fori_loop(0, _NVIS, samp, 0)

    def msk(j, _):
        r = midx_ref[j]
        src_ref[r // _RPW, r % _RPW] = -2
        return 0

    lax.fori_loop(0, _NMASK, msk, 0)


def _sc_body(x_hbm, src_hbm, w0_hbm, b_hbm, out_hbm,
             srcb, qx, gx, qr, xbuf, sbuf, zbuf, wbuf, bbuf,
             semg, semx, semc):
    cid = lax.axis_index("c")
    sid = lax.axis_index("s")
    wid = sid * 2 + cid
    pltpu.sync_copy(src_hbm.at[wid], srcb)
    pltpu.sync_copy(w0_hbm, wbuf)
    pltpu.sync_copy(b_hbm.at[pl.ds(0, 16)], bbuf)
    acc = jnp.zeros((16,), jnp.float32)
    for k in range(_EMBED // 16):
        acc = acc + wbuf[pl.ds(k * 16, 16)]
    lanes = lax.iota(jnp.int32, 16)
    wsum = acc[0]
    for k in range(1, 16):
        wsum = wsum + acc[k]
    b0 = bbuf[...][0]
    s_val = wsum * np.float32(127.0 / 255.0) + b0
    sv = jnp.full((16,), s_val, jnp.float32)
    zv = jnp.zeros((16,), jnp.float32)

    def fill(row, _):
        for k in range(_EMBED // 16):
            sbuf[row, pl.ds(k * 16, 16)] = sv
            zbuf[row, pl.ds(k * 16, 16)] = zv
        return 0

    lax.fori_loop(0, 32, fill, 0)

    def row_body(i, carry):
        cnt, cslot = carry
        src_i = srcb[pl.ds(i, 16)][0]
        r = wid * _RPW + i
        is_const = jnp.logical_or(src_i == -1, src_i == -2)

        @pl.when(src_i >= 0)
        def _():
            for h in range(2):
                for k in range(2):
                    qx[pl.ds(k * 16, 16)] = (
                        (lanes + (h * 32 + k * 16)) * _NROWS + r
                    )
                    gx[pl.ds(k * 16, 16)] = (
                        (lanes + (h * 32 + k * 16)) * (_NVIS + 1) + src_i
                    )
                pltpu.async_copy(x_hbm.at[gx], xbuf, semg).wait()
                pltpu.async_copy(xbuf, out_hbm.at[qx], semx).wait()

        ic = is_const.astype(jnp.int32)
        out0 = cnt
        for h in range(2):
            waited = jnp.logical_and(is_const, out0 >= 4)

            @pl.when(waited)
            def _():
                pltpu.make_async_copy(sbuf, out_hbm.at[qr.at[0]], semc).wait()

            slot = lax.rem(cslot + h, 4)

            @pl.when(is_const)
            def _(slot=slot, h=h):
                for k in range(2):
                    qr[slot, pl.ds(k * 16, 16)] = (
                        (lanes + (h * 32 + k * 16)) * _NROWS + r
                    )

                @pl.when(src_i == -2)
                def _():
                    pltpu.async_copy(sbuf, out_hbm.at[qr.at[slot]], semc)

                @pl.when(src_i == -1)
                def _():
                    pltpu.async_copy(zbuf, out_hbm.at[qr.at[slot]], semc)

            out0 = out0 - waited.astype(jnp.int32) + ic

        return (out0, lax.rem(cslot + 2 * ic, 4))

    cnt, _unused = lax.fori_loop(0, _RPW, row_body, (0, 0))

    def drain(k, c):
        pltpu.make_async_copy(sbuf, out_hbm.at[qr.at[0]], semc).wait()
        return c

    lax.fori_loop(0, cnt, drain, 0)


def kernel(x, sample_index, mask_index, W, b):
    src2d = pl.pallas_call(
        _build_maps,
        in_specs=[
            pl.BlockSpec(memory_space=pltpu.SMEM),
            pl.BlockSpec(memory_space=pltpu.SMEM),
        ],
        out_specs=pl.BlockSpec(memory_space=pltpu.SMEM),
        out_shape=jax.ShapeDtypeStruct((_NW, _SP), jnp.int32),
    )(sample_index, mask_index)

    x2d = jnp.reshape(x, (_B * (1 + _NVIS), _EMBED))
    w0 = jnp.reshape(W[0], (_EMBED,))

    mesh = plsc.VectorSubcoreMesh(core_axis_name="c", subcore_axis_name="s")
    out2 = pl.kernel(
        _sc_body,
        out_type=jax.ShapeDtypeStruct((_B * _NROWS, _EMBED), jnp.float32),
        mesh=mesh,
        scratch_types=[
            pltpu.VMEM((_SP,), jnp.int32),
            pltpu.VMEM((32,), jnp.int32),
            pltpu.VMEM((32,), jnp.int32),
            pltpu.VMEM((4, 32), jnp.int32),
            pltpu.VMEM((32, _EMBED), jnp.float32),
            pltpu.VMEM((32, _EMBED), jnp.float32),
            pltpu.VMEM((32, _EMBED), jnp.float32),
            pltpu.VMEM((_EMBED,), jnp.float32),
            pltpu.VMEM((16,), jnp.float32),
            pltpu.SemaphoreType.DMA,
            pltpu.SemaphoreType.DMA,
            pltpu.SemaphoreType.DMA,
        ],
    )(x2d, src2d, w0, b)

    return jnp.reshape(out2, (_B, _NROWS, _EMBED))
